# baseline (speedup 1.0000x reference)
import jax
import jax.numpy as jnp
from jax import lax
from jax.experimental import pallas as pl
from jax.experimental.pallas import tpu as pltpu

MASKS_SIZE = 64
GRID_W = 6
C_CHUNK = 8


def _mul_body(m_ref, x_ref, o_ref):
    b = pl.program_id(0)
    row = lax.broadcasted_iota(jnp.int32, (384, 384), 0) // MASKS_SIZE
    col = lax.broadcasted_iota(jnp.int32, (384, 384), 1) // MASKS_SIZE
    cell = row * GRID_W + col
    keep = jnp.ones((384, 384), dtype=jnp.bool_)
    for k in range(4):
        keep = jnp.logical_and(keep, cell != m_ref[b, k])
    m = keep.astype(jnp.float32)[None, None, :, :]
    o_ref[...] = x_ref[...] * m


def kernel(x, m_indices):
    b, c, h, w = x.shape
    grid = (b, c // C_CHUNK)
    return pl.pallas_call(
        _mul_body,
        grid=grid,
        in_specs=[
            pl.BlockSpec(memory_space=pltpu.SMEM),
            pl.BlockSpec(
                (1, C_CHUNK, h, w),
                lambda i, j: (i, j, 0, 0),
            ),
        ],
        out_specs=pl.BlockSpec(
            (1, C_CHUNK, h, w),
            lambda i, j: (i, j, 0, 0),
        ),
        out_shape=jax.ShapeDtypeStruct(x.shape, x.dtype),
    )(m_indices, x)


# speedup vs baseline: 1.0130x; 1.0130x over previous
import jax
import jax.numpy as jnp
from jax import lax
from jax.experimental import pallas as pl
from jax.experimental.pallas import tpu as pltpu

MASKS_SIZE = 64
GRID_W = 6
C_CHUNK = 24


def _mul_body(m_ref, x_ref, o_ref):
    b = pl.program_id(0)
    row = lax.broadcasted_iota(jnp.int32, (384, 384), 0) // MASKS_SIZE
    col = lax.broadcasted_iota(jnp.int32, (384, 384), 1) // MASKS_SIZE
    cell = row * GRID_W + col
    keep = jnp.ones((384, 384), dtype=jnp.bool_)
    for k in range(4):
        keep = jnp.logical_and(keep, cell != m_ref[b, k])
    m = keep.astype(jnp.float32)[None, None, :, :]
    o_ref[...] = x_ref[...] * m


def kernel(x, m_indices):
    b, c, h, w = x.shape
    grid = (b, c // C_CHUNK)
    return pl.pallas_call(
        _mul_body,
        grid=grid,
        in_specs=[
            pl.BlockSpec(memory_space=pltpu.SMEM),
            pl.BlockSpec(
                (1, C_CHUNK, h, w),
                lambda i, j: (i, j, 0, 0),
            ),
        ],
        out_specs=pl.BlockSpec(
            (1, C_CHUNK, h, w),
            lambda i, j: (i, j, 0, 0),
        ),
        out_shape=jax.ShapeDtypeStruct(x.shape, x.dtype),
    )(m_indices, x)
